# Initial kernel scaffold; baseline (speedup 1.0000x reference)
import jax
import jax.numpy as jnp
from jax.experimental import pallas as pl


def kernel(close_er, y, max_dis, margin):
    raise NotImplementedError("write your pallas kernel here")


# speedup vs baseline: 6.4875x; 6.4875x over previous
import functools

import jax
import jax.numpy as jnp
from jax import lax
from jax.experimental import pallas as pl
from jax.experimental.pallas import tpu as pltpu
from jax.experimental.pallas import tpu_sc as plsc

_NC = 2
_NS = 16
_NW = _NC * _NS
_L = 16

_LO = 1.1920930376163597e-07
_HI = 16.118095651262775


def _per_sample_loss(t, marg_v):
    u = t + marg_v
    e = jnp.exp(-jnp.abs(u))
    s = e / (e + 2.0)
    s2 = s * s
    p = 2.0 * s * (1.0 + s2 * (1.0 / 3.0 + s2 * (0.2 + s2 * (1.0 / 7.0 + s2 * (1.0 / 9.0)))))
    ps = jnp.maximum(u, 0.0) + p
    return jnp.clip(ps, _LO, _HI)


@functools.lru_cache(maxsize=None)
def _build(n, c):
    assert c % _L == 0
    per_w = -(-n // (_NW * 128)) * 128
    npad = per_w * _NW
    ch = per_w // 128
    cb = c // _L
    mesh = plsc.VectorSubcoreMesh(core_axis_name="c", subcore_axis_name="s")
    params = pltpu.CompilerParams(
        use_tc_tiling_on_sc=False, needs_layout_passes=False)

    @functools.partial(
        pl.kernel,
        out_type=jax.ShapeDtypeStruct((_NW, 2 * c), jnp.float32),
        mesh=mesh,
        compiler_params=params,
        scratch_types=[
            pltpu.VMEM((per_w,), jnp.int32),
            pltpu.VMEM((ch, 128), jnp.int32),
            pltpu.VMEM((ch, 128), jnp.float32),
            pltpu.VMEM((c,), jnp.float32),
            pltpu.VMEM((_L,), jnp.float32),
            pltpu.VMEM((_L * c,), jnp.float32),
            pltpu.VMEM((_L * c,), jnp.float32),
            pltpu.VMEM((2 * c,), jnp.float32),
            pltpu.SemaphoreType.DMA,
        ],
    )
    def k1(close_hbm, y_hbm, md_hbm, marg_hbm, part_hbm,
           y_v, idx_v, gath_v, md_v, marg_v, sbin_v, cbin_v, fold_v, sem):
        wid = lax.axis_index("s") * _NC + lax.axis_index("c")
        base = wid * per_w
        pltpu.sync_copy(y_hbm.at[pl.ds(base, per_w)], y_v)
        pltpu.sync_copy(md_hbm, md_v)
        pltpu.sync_copy(marg_hbm, marg_v)
        iota = lax.iota(jnp.int32, _L)
        zf = jnp.zeros((_L,), jnp.float32)
        onef = jnp.full((_L,), 1.0, jnp.float32)
        marg = marg_v[...]

        def zero_body(j, _):
            sbin_v[pl.ds(j * _L, _L)] = zf
            cbin_v[pl.ds(j * _L, _L)] = zf
            return 0
        lax.fori_loop(0, _L * c // _L, zero_body, 0)

        def idx_body(r, _):
            for cj in range(8):
                off = r * 128 + cj * _L
                yv = y_v[pl.ds(off, _L)]
                valid = yv < c
                yc = jnp.minimum(yv, c - 1)
                flat = jnp.where(valid, (base + off + iota) * c + yc, 0)
                idx_v[r, pl.ds(cj * _L, _L)] = flat
            return 0
        lax.fori_loop(0, ch, idx_body, 0)

        def fire(r, _):
            pltpu.make_async_copy(close_hbm.at[idx_v.at[r]], gath_v.at[r], sem).start()
            return 0
        lax.fori_loop(0, ch, fire, 0)

        def drain(r, _):
            pltpu.make_async_copy(close_hbm.at[idx_v.at[r]], gath_v.at[r], sem).wait()
            return 0
        lax.fori_loop(0, ch, drain, 0)

        def comp_body(r, _):
            for cj in range(8):
                off = r * 128 + cj * _L
                g = gath_v[r, pl.ds(cj * _L, _L)]
                yv = y_v[pl.ds(off, _L)]
                valid = yv < c
                yc = jnp.minimum(yv, c - 1)
                md = plsc.load_gather(md_v, [yc])
                ps = _per_sample_loss(md - g, marg)
                bidx = iota * c + yc
                plsc.addupdate_scatter(sbin_v, [bidx], ps, mask=valid)
                plsc.addupdate_scatter(cbin_v, [bidx], onef, mask=valid)
            return 0
        lax.fori_loop(0, ch, comp_body, 0)

        def fold_body(b, _):
            accs = zf
            accc = zf
            for l in range(_L):
                accs = accs + sbin_v[pl.ds(l * c + b * _L, _L)]
                accc = accc + cbin_v[pl.ds(l * c + b * _L, _L)]
            fold_v[pl.ds(b * _L, _L)] = accs
            fold_v[pl.ds(c + b * _L, _L)] = accc
            return 0
        lax.fori_loop(0, cb, fold_body, 0)
        pltpu.sync_copy(fold_v, part_hbm.at[wid])

    @functools.partial(
        pl.kernel,
        out_type=jax.ShapeDtypeStruct((_L,), jnp.float32),
        mesh=mesh,
        compiler_params=params,
        scratch_types=[
            pltpu.VMEM((_NW, 2 * c), jnp.float32),
            pltpu.VMEM((_L,), jnp.float32),
        ],
    )
    def k2(part_hbm, out_hbm, buf_v, out_v):
        wid = lax.axis_index("s") * _NC + lax.axis_index("c")

        @pl.when(wid == 0)
        def _():
            pltpu.sync_copy(part_hbm, buf_v)
            zf = jnp.zeros((_L,), jnp.float32)
            onef = jnp.full((_L,), 1.0, jnp.float32)

            def red_body(b, car):
                macc, jacc = car
                accs = zf
                accc = zf
                for w in range(_NW):
                    accs = accs + buf_v[w, pl.ds(b * _L, _L)]
                    accc = accc + buf_v[w, pl.ds(c + b * _L, _L)]
                ne = accc > 0.0
                mean = jnp.where(ne, accs / jnp.maximum(accc, 1.0), zf)
                return macc + mean, jacc + jnp.where(ne, onef, zf)

            macc, jacc = lax.fori_loop(0, cb, red_body, (zf, zf))
            out_v[...] = (zf + jnp.sum(macc)) / jnp.maximum(zf + jnp.sum(jacc), 1.0)
            pltpu.sync_copy(out_v, out_hbm)

    def run(close_er, y, max_dis, margin):
        close_flat = close_er.reshape(-1)
        y_pad = jnp.concatenate(
            [y.astype(jnp.int32), jnp.full((npad - n,), c, jnp.int32)])
        marg_arr = jnp.broadcast_to(margin.astype(jnp.float32), (_L,))
        part = k1(close_flat, y_pad, max_dis, marg_arr)
        return k2(part)[0]

    return run


def kernel(close_er, y, max_dis, margin):
    n, c = close_er.shape
    return _build(n, c)(close_er, y, max_dis, jnp.asarray(margin))


# speedup vs baseline: 10.8220x; 1.6681x over previous
import functools

import jax
import jax.numpy as jnp
from jax import lax
from jax.experimental import pallas as pl
from jax.experimental.pallas import tpu as pltpu
from jax.experimental.pallas import tpu_sc as plsc

_NC = 2
_NS = 16
_NW = _NC * _NS
_L = 16
_B = 128

_LO = 1.1920930376163597e-07
_HI = 16.118095651262775


def _per_sample_loss(t, marg_v):
    u = t + marg_v
    e = jnp.exp(-jnp.abs(u))
    s = e / (e + 2.0)
    s2 = s * s
    p = 2.0 * s * (1.0 + s2 * (1.0 / 3.0 + s2 * (0.2 + s2 * (1.0 / 7.0 + s2 * (1.0 / 9.0)))))
    ps = jnp.maximum(u, 0.0) + p
    return jnp.clip(ps, _LO, _HI)


@functools.lru_cache(maxsize=None)
def _build(n, c):
    assert c % 128 == 0 and n % 8 == 0 and n >= _B * _NW
    tpr = c // 128
    per_w = -(-n // (_NW * 8)) * 8
    ypad = per_w * _NW
    cb = c // _L
    mesh = plsc.VectorSubcoreMesh(core_axis_name="c", subcore_axis_name="s")
    params = pltpu.CompilerParams(
        use_tc_tiling_on_sc=True, needs_layout_passes=False)

    @functools.partial(
        pl.kernel,
        out_type=jax.ShapeDtypeStruct((_NW * 2 * c,), jnp.float32),
        mesh=mesh,
        compiler_params=params,
        scratch_types=[
            pltpu.VMEM((per_w,), jnp.int32),
            pltpu.VMEM((2, _B, c), jnp.float32),
            pltpu.VMEM((c,), jnp.float32),
            pltpu.VMEM((_L,), jnp.float32),
            pltpu.VMEM((_L * c,), jnp.float32),
            pltpu.VMEM((_L * c,), jnp.float32),
            pltpu.VMEM((2 * c,), jnp.float32),
            pltpu.SemaphoreType.DMA,
            pltpu.SemaphoreType.DMA,
        ],
    )
    def k1(close_hbm, y_hbm, md_hbm, marg_hbm, part_hbm,
           y_v, blk_v, md_v, marg_v, sbin_v, cbin_v, fold_v, sem0, sem1):
        wid = lax.axis_index("s") * _NC + lax.axis_index("c")
        base = wid * per_w
        rows_w = jnp.minimum(per_w, n - base)
        nfull = rows_w // _B
        thresh = base + nfull * _B
        nblk = nfull + 1
        pltpu.sync_copy(y_hbm.at[pl.ds(base, per_w)], y_v)
        pltpu.sync_copy(md_hbm, md_v)
        pltpu.sync_copy(marg_hbm, marg_v)
        iota = lax.iota(jnp.int32, _L)
        zf = jnp.zeros((_L,), jnp.float32)
        onef = jnp.full((_L,), 1.0, jnp.float32)
        marg = marg_v[...]

        def zero_body(j, _):
            sbin_v[pl.ds(j * _L, _L)] = zf
            cbin_v[pl.ds(j * _L, _L)] = zf
            return 0
        lax.fori_loop(0, _L * c // _L, zero_body, 0)

        def r0_of(k):
            r0 = jnp.where(k < nfull, base + k * _B, base + rows_w - _B)
            return pl.multiple_of(r0, 8)

        def copy_of(k, parity):
            sem = sem0 if parity == 0 else sem1
            return pltpu.make_async_copy(
                close_hbm.at[pl.ds(r0_of(k), _B), :], blk_v.at[parity], sem)

        def start(k, parity):
            @pl.when(k % 2 == parity)
            def _():
                copy_of(k, parity).start()

        copy_of(0, 0).start()

        def blk_body(k, _):
            @pl.when(k + 1 < nblk)
            def _():
                start(k + 1, 0)
                start(k + 1, 1)

            @pl.when(k % 2 == 0)
            def _():
                copy_of(k, 0).wait()

            @pl.when(k % 2 == 1)
            def _():
                copy_of(k, 1).wait()
            r0 = r0_of(k)
            buf = blk_v.at[k % 2]
            for q in range(_B // _L):
                row = r0 + q * _L + iota
                keep = row >= jnp.where(k < nfull, 0, thresh)
                yv = y_v[pl.ds(r0 - base + q * _L, _L)]
                rl = q * _L + iota
                g = plsc.load_gather(buf, [rl, yv])
                md = plsc.load_gather(md_v, [yv])
                ps = _per_sample_loss(md - g, marg)
                bidx = iota * c + yv
                plsc.addupdate_scatter(sbin_v, [bidx], ps, mask=keep)
                plsc.addupdate_scatter(cbin_v, [bidx], onef, mask=keep)
            return 0
        lax.fori_loop(0, nblk, blk_body, 0)

        def fold_body(b, _):
            accs = zf
            accc = zf
            for l in range(_L):
                accs = accs + sbin_v[pl.ds(l * c + b * _L, _L)]
                accc = accc + cbin_v[pl.ds(l * c + b * _L, _L)]
            fold_v[pl.ds(b * _L, _L)] = accs
            fold_v[pl.ds(c + b * _L, _L)] = accc
            return 0
        lax.fori_loop(0, cb, fold_body, 0)
        pltpu.sync_copy(fold_v, part_hbm.at[pl.ds(wid * 2 * c, 2 * c)])

    @functools.partial(
        pl.kernel,
        out_type=jax.ShapeDtypeStruct((_L,), jnp.float32),
        mesh=mesh,
        compiler_params=params,
        scratch_types=[
            pltpu.VMEM((_NW * 2 * c,), jnp.float32),
            pltpu.VMEM((_L,), jnp.float32),
        ],
    )
    def k2(part_hbm, out_hbm, buf_v, out_v):
        wid = lax.axis_index("s") * _NC + lax.axis_index("c")

        @pl.when(wid == 0)
        def _():
            pltpu.sync_copy(part_hbm, buf_v)
            zf = jnp.zeros((_L,), jnp.float32)
            onef = jnp.full((_L,), 1.0, jnp.float32)

            def red_body(b, car):
                macc, jacc = car
                accs = zf
                accc = zf
                for w in range(_NW):
                    accs = accs + buf_v[pl.ds(w * 2 * c + b * _L, _L)]
                    accc = accc + buf_v[pl.ds(w * 2 * c + c + b * _L, _L)]
                ne = accc > 0.0
                mean = jnp.where(ne, accs / jnp.maximum(accc, 1.0), zf)
                return macc + mean, jacc + jnp.where(ne, onef, zf)

            macc, jacc = lax.fori_loop(0, cb, red_body, (zf, zf))
            out_v[...] = (zf + jnp.sum(macc)) / jnp.maximum(zf + jnp.sum(jacc), 1.0)
            pltpu.sync_copy(out_v, out_hbm)

    def run(close_er, y, max_dis, margin):
        y_pad = jnp.concatenate(
            [y.astype(jnp.int32), jnp.zeros((ypad - n,), jnp.int32)])
        marg_arr = jnp.broadcast_to(margin.astype(jnp.float32), (_L,))
        part = k1(close_er, y_pad, max_dis, marg_arr)
        return k2(part)[0]

    return run


def kernel(close_er, y, max_dis, margin):
    n, c = close_er.shape
    return _build(n, c)(close_er, y, max_dis, jnp.asarray(margin))


# speedup vs baseline: 10.8251x; 1.0003x over previous
import functools

import jax
import jax.numpy as jnp
from jax import lax
from jax.experimental import pallas as pl
from jax.experimental.pallas import tpu as pltpu
from jax.experimental.pallas import tpu_sc as plsc

_NC = 2
_NS = 16
_NW = _NC * _NS
_L = 16
_B = 192

_LO = 1.1920930376163597e-07
_HI = 16.118095651262775


def _per_sample_loss(t, marg_v):
    u = t + marg_v
    e = jnp.exp(-jnp.abs(u))
    s = e / (e + 2.0)
    s2 = s * s
    p = 2.0 * s * (1.0 + s2 * (1.0 / 3.0 + s2 * (0.2 + s2 * (1.0 / 7.0 + s2 * (1.0 / 9.0)))))
    ps = jnp.maximum(u, 0.0) + p
    return jnp.clip(ps, _LO, _HI)


@functools.lru_cache(maxsize=None)
def _build(n, c):
    per_w = -(-n // (_NW * 8)) * 8
    assert c % 128 == 0 and n % 8 == 0
    assert n - (_NW - 1) * per_w >= _B
    cb = c // _L
    mesh = plsc.VectorSubcoreMesh(core_axis_name="c", subcore_axis_name="s")
    params = pltpu.CompilerParams(
        use_tc_tiling_on_sc=True, needs_layout_passes=False)

    @functools.partial(
        pl.kernel,
        out_type=jax.ShapeDtypeStruct((_NW * 2 * c,), jnp.float32),
        mesh=mesh,
        compiler_params=params,
        scratch_types=[
            pltpu.VMEM((per_w,), jnp.int32),
            pltpu.VMEM((2, _B, c), jnp.float32),
            pltpu.VMEM((c,), jnp.float32),
            pltpu.VMEM((_L,), jnp.float32),
            pltpu.VMEM((_L * c,), jnp.float32),
            pltpu.VMEM((_L * c,), jnp.float32),
            pltpu.VMEM((2 * c,), jnp.float32),
            pltpu.SemaphoreType.DMA,
            pltpu.SemaphoreType.DMA,
        ],
    )
    def k1(close_hbm, y_hbm, md_hbm, marg_hbm, part_hbm,
           y_v, blk_v, md_v, marg_v, sbin_v, cbin_v, fold_v, sem0, sem1):
        wid = lax.axis_index("s") * _NC + lax.axis_index("c")
        base = wid * per_w
        rows_w = jnp.minimum(per_w, n - base)
        nfull = rows_w // _B
        thresh = base + nfull * _B
        nblk = nfull + 1
        ybase = pl.multiple_of(jnp.minimum(base, n - per_w), 8)
        pltpu.sync_copy(y_hbm.at[pl.ds(ybase, per_w)], y_v)
        pltpu.sync_copy(md_hbm, md_v)
        pltpu.sync_copy(marg_hbm, marg_v)
        iota = lax.iota(jnp.int32, _L)
        zf = jnp.zeros((_L,), jnp.float32)
        onef = jnp.full((_L,), 1.0, jnp.float32)
        marg = marg_v[...]

        def zero_body(j, _):
            sbin_v[pl.ds(j * _L, _L)] = zf
            cbin_v[pl.ds(j * _L, _L)] = zf
            return 0
        lax.fori_loop(0, _L * c // _L, zero_body, 0)

        def r0_of(k):
            r0 = jnp.where(k < nfull, base + k * _B, base + rows_w - _B)
            return pl.multiple_of(r0, 8)

        def copy_of(k, parity):
            sem = sem0 if parity == 0 else sem1
            return pltpu.make_async_copy(
                close_hbm.at[pl.ds(r0_of(k), _B), :], blk_v.at[parity], sem)

        def start(k, parity):
            @pl.when(k % 2 == parity)
            def _():
                copy_of(k, parity).start()

        copy_of(0, 0).start()

        def blk_body(k, _):
            @pl.when(k + 1 < nblk)
            def _():
                start(k + 1, 0)
                start(k + 1, 1)

            @pl.when(k % 2 == 0)
            def _():
                copy_of(k, 0).wait()

            @pl.when(k % 2 == 1)
            def _():
                copy_of(k, 1).wait()
            r0 = r0_of(k)
            buf = blk_v.at[k % 2]
            for q in range(_B // _L):
                row = r0 + q * _L + iota
                keep = row >= jnp.where(k < nfull, 0, thresh)
                yv = y_v[pl.ds(r0 - ybase + q * _L, _L)]
                rl = q * _L + iota
                g = plsc.load_gather(buf, [rl, yv])
                md = plsc.load_gather(md_v, [yv])
                ps = _per_sample_loss(md - g, marg)
                bidx = iota * c + yv
                plsc.addupdate_scatter(sbin_v, [bidx], ps, mask=keep)
                plsc.addupdate_scatter(cbin_v, [bidx], onef, mask=keep)
            return 0
        lax.fori_loop(0, nblk, blk_body, 0)

        def fold_body(b, _):
            accs = zf
            accc = zf
            for l in range(_L):
                accs = accs + sbin_v[pl.ds(l * c + b * _L, _L)]
                accc = accc + cbin_v[pl.ds(l * c + b * _L, _L)]
            fold_v[pl.ds(b * _L, _L)] = accs
            fold_v[pl.ds(c + b * _L, _L)] = accc
            return 0
        lax.fori_loop(0, cb, fold_body, 0)
        pltpu.sync_copy(fold_v, part_hbm.at[pl.ds(wid * 2 * c, 2 * c)])

    @functools.partial(
        pl.kernel,
        out_type=jax.ShapeDtypeStruct((_L,), jnp.float32),
        mesh=mesh,
        compiler_params=params,
        scratch_types=[
            pltpu.VMEM((_NW * 2 * c,), jnp.float32),
            pltpu.VMEM((_L,), jnp.float32),
        ],
    )
    def k2(part_hbm, out_hbm, buf_v, out_v):
        wid = lax.axis_index("s") * _NC + lax.axis_index("c")

        @pl.when(wid == 0)
        def _():
            pltpu.sync_copy(part_hbm, buf_v)
            zf = jnp.zeros((_L,), jnp.float32)
            onef = jnp.full((_L,), 1.0, jnp.float32)

            def red_body(b, car):
                macc, jacc = car
                accs = zf
                accc = zf
                for w in range(_NW):
                    accs = accs + buf_v[pl.ds(w * 2 * c + b * _L, _L)]
                    accc = accc + buf_v[pl.ds(w * 2 * c + c + b * _L, _L)]
                ne = accc > 0.0
                mean = jnp.where(ne, accs / jnp.maximum(accc, 1.0), zf)
                return macc + mean, jacc + jnp.where(ne, onef, zf)

            macc, jacc = lax.fori_loop(0, cb, red_body, (zf, zf))
            out_v[...] = (zf + jnp.sum(macc)) / jnp.maximum(zf + jnp.sum(jacc), 1.0)
            pltpu.sync_copy(out_v, out_hbm)

    def run(close_er, y, max_dis, margin):
        marg_arr = jnp.broadcast_to(margin.astype(jnp.float32), (_L,))
        part = k1(close_er, y.astype(jnp.int32), max_dis, marg_arr)
        return k2(part)[0]

    return run


def kernel(close_er, y, max_dis, margin):
    n, c = close_er.shape
    return _build(n, c)(close_er, y, max_dis, jnp.asarray(margin))


# speedup vs baseline: 11.6539x; 1.0766x over previous
import functools

import jax
import jax.numpy as jnp
from jax import lax
from jax.experimental import pallas as pl
from jax.experimental.pallas import tpu as pltpu
from jax.experimental.pallas import tpu_sc as plsc

_NC = 2
_NS = 16
_NW = _NC * _NS
_L = 16
_B = 128
_NBUF = 3

_LO = 1.1920930376163597e-07
_HI = 16.118095651262775


def _per_sample_loss(u):
    e = jnp.exp(-jnp.abs(u))
    s = e / (e + 2.0)
    s2 = s * s
    p = 2.0 * s * (1.0 + s2 * (1.0 / 3.0 + s2 * (0.2 + s2 * (1.0 / 7.0 + s2 * (1.0 / 9.0)))))
    ps = jnp.maximum(u, 0.0) + p
    return jnp.clip(ps, _LO, _HI)


@functools.lru_cache(maxsize=None)
def _build(n, c):
    per_w = -(-n // (_NW * 8)) * 8
    assert c % 128 == 0 and n % 8 == 0
    assert n - (_NW - 1) * per_w >= _B
    cb = c // _L
    mesh = plsc.VectorSubcoreMesh(core_axis_name="c", subcore_axis_name="s")
    params = pltpu.CompilerParams(
        use_tc_tiling_on_sc=True, needs_layout_passes=False)

    @functools.partial(
        pl.kernel,
        out_type=jax.ShapeDtypeStruct((_NW * 2 * c,), jnp.float32),
        mesh=mesh,
        compiler_params=params,
        scratch_types=[
            pltpu.VMEM((per_w,), jnp.int32),
            pltpu.VMEM((_NBUF, _B, c), jnp.float32),
            pltpu.VMEM((c,), jnp.float32),
            pltpu.VMEM((_L * c,), jnp.float32),
            pltpu.VMEM((_L * c,), jnp.float32),
            pltpu.VMEM((2 * c,), jnp.float32),
            [pltpu.SemaphoreType.DMA] * _NBUF,
        ],
    )
    def k1(close_hbm, y_hbm, md_hbm, part_hbm,
           y_v, blk_v, md_v, sbin_v, cbin_v, fold_v, sems):
        wid = lax.axis_index("s") * _NC + lax.axis_index("c")
        base = wid * per_w
        rows_w = jnp.minimum(per_w, n - base)
        nfull = rows_w // _B
        thresh = base + nfull * _B
        nblk = nfull + 1
        ybase = pl.multiple_of(jnp.minimum(base, n - per_w), 8)
        pltpu.sync_copy(y_hbm.at[pl.ds(ybase, per_w)], y_v)
        pltpu.sync_copy(md_hbm, md_v)
        iota = lax.iota(jnp.int32, _L)
        zf = jnp.zeros((_L,), jnp.float32)
        onef = jnp.full((_L,), 1.0, jnp.float32)

        def zero_body(j, _):
            sbin_v[pl.ds(j * _L, _L)] = zf
            cbin_v[pl.ds(j * _L, _L)] = zf
            return 0
        lax.fori_loop(0, _L * c // _L, zero_body, 0)

        def r0_of(k):
            r0 = jnp.where(k < nfull, base + k * _B, base + rows_w - _B)
            return pl.multiple_of(r0, 8)

        def copy_of(k, slot):
            return pltpu.make_async_copy(
                close_hbm.at[pl.ds(r0_of(k), _B), :], blk_v.at[slot], sems[slot])

        def start(k):
            for slot in range(_NBUF):
                @pl.when(k % _NBUF == slot)
                def _():
                    copy_of(k, slot).start()

        for b in range(_NBUF - 1):
            @pl.when(b < nblk)
            def _():
                copy_of(b, b).start()

        def blk_body(k, _):
            @pl.when(k + _NBUF - 1 < nblk)
            def _():
                start(k + _NBUF - 1)
            for slot in range(_NBUF):
                @pl.when(k % _NBUF == slot)
                def _():
                    copy_of(k, slot).wait()
            r0 = r0_of(k)
            buf = blk_v.at[k % _NBUF]
            for q in range(_B // _L):
                row = r0 + q * _L + iota
                keep = row >= jnp.where(k < nfull, 0, thresh)
                yv = y_v[pl.ds(r0 - ybase + q * _L, _L)]
                rl = q * _L + iota
                g = plsc.load_gather(buf, [rl, yv])
                md = plsc.load_gather(md_v, [yv])
                ps = _per_sample_loss(md - g)
                bidx = iota * c + yv
                plsc.addupdate_scatter(sbin_v, [bidx], ps, mask=keep)
                plsc.addupdate_scatter(cbin_v, [bidx], onef, mask=keep)
            return 0
        lax.fori_loop(0, nblk, blk_body, 0)

        def fold_body(b, _):
            accs = zf
            accc = zf
            for l in range(_L):
                accs = accs + sbin_v[pl.ds(l * c + b * _L, _L)]
                accc = accc + cbin_v[pl.ds(l * c + b * _L, _L)]
            fold_v[pl.ds(b * _L, _L)] = accs
            fold_v[pl.ds(c + b * _L, _L)] = accc
            return 0
        lax.fori_loop(0, cb, fold_body, 0)
        pltpu.sync_copy(fold_v, part_hbm.at[pl.ds(wid * 2 * c, 2 * c)])

    @functools.partial(
        pl.kernel,
        out_type=jax.ShapeDtypeStruct((_L,), jnp.float32),
        mesh=mesh,
        compiler_params=params,
        scratch_types=[
            pltpu.VMEM((_NW * 2 * c,), jnp.float32),
            pltpu.VMEM((_L,), jnp.float32),
        ],
    )
    def k2(part_hbm, out_hbm, buf_v, out_v):
        wid = lax.axis_index("s") * _NC + lax.axis_index("c")

        @pl.when(wid == 0)
        def _():
            pltpu.sync_copy(part_hbm, buf_v)
            zf = jnp.zeros((_L,), jnp.float32)
            onef = jnp.full((_L,), 1.0, jnp.float32)

            def red_body(b, car):
                macc, jacc = car
                accs = zf
                accc = zf
                for w in range(_NW):
                    accs = accs + buf_v[pl.ds(w * 2 * c + b * _L, _L)]
                    accc = accc + buf_v[pl.ds(w * 2 * c + c + b * _L, _L)]
                ne = accc > 0.0
                mean = jnp.where(ne, accs / jnp.maximum(accc, 1.0), zf)
                return macc + mean, jacc + jnp.where(ne, onef, zf)

            macc, jacc = lax.fori_loop(0, cb, red_body, (zf, zf))
            out_v[...] = (zf + jnp.sum(macc)) / jnp.maximum(zf + jnp.sum(jacc), 1.0)
            pltpu.sync_copy(out_v, out_hbm)

    def run(close_er, y, max_dis, margin):
        md_plus = max_dis + margin.astype(jnp.float32)
        part = k1(close_er, y.astype(jnp.int32), md_plus)
        return k2(part)[0]

    return run


def kernel(close_er, y, max_dis, margin):
    n, c = close_er.shape
    return _build(n, c)(close_er, y, max_dis, jnp.asarray(margin))


# speedup vs baseline: 12.3297x; 1.0580x over previous
import functools

import jax
import jax.numpy as jnp
from jax import lax
from jax.experimental import pallas as pl
from jax.experimental.pallas import tpu as pltpu
from jax.experimental.pallas import tpu_sc as plsc

_NC = 2
_NS = 16
_NW = _NC * _NS
_L = 16
_B = 96
_NBUF = 4

_LO = 1.1920930376163597e-07
_HI = 16.118095651262775


def _per_sample_loss(u):
    e = jnp.exp(-jnp.abs(u))
    s = e / (e + 2.0)
    s2 = s * s
    p = 2.0 * s * (1.0 + s2 * (1.0 / 3.0 + s2 * (0.2 + s2 * (1.0 / 7.0 + s2 * (1.0 / 9.0)))))
    ps = jnp.maximum(u, 0.0) + p
    return jnp.clip(ps, _LO, _HI)


@functools.lru_cache(maxsize=None)
def _build(n, c):
    per_w = -(-n // (_NW * 8)) * 8
    assert c % 128 == 0 and n % 8 == 0
    assert n - (_NW - 1) * per_w >= _B
    cb = c // _L
    mesh = plsc.VectorSubcoreMesh(core_axis_name="c", subcore_axis_name="s")
    params = pltpu.CompilerParams(
        use_tc_tiling_on_sc=True, needs_layout_passes=False)

    @functools.partial(
        pl.kernel,
        out_type=jax.ShapeDtypeStruct((_NW * 2 * c,), jnp.float32),
        mesh=mesh,
        compiler_params=params,
        scratch_types=[
            pltpu.VMEM((per_w,), jnp.int32),
            pltpu.VMEM((_NBUF, _B, c), jnp.float32),
            pltpu.VMEM((c,), jnp.float32),
            pltpu.VMEM((_L * c,), jnp.float32),
            pltpu.VMEM((_L * c,), jnp.float32),
            pltpu.VMEM((2 * c,), jnp.float32),
            [pltpu.SemaphoreType.DMA] * _NBUF,
        ],
    )
    def k1(close_hbm, y_hbm, md_hbm, part_hbm,
           y_v, blk_v, md_v, sbin_v, cbin_v, fold_v, sems):
        wid = lax.axis_index("s") * _NC + lax.axis_index("c")
        base = wid * per_w
        rows_w = jnp.minimum(per_w, n - base)
        nfull = rows_w // _B
        thresh = base + nfull * _B
        nblk = nfull + 1
        ybase = pl.multiple_of(jnp.minimum(base, n - per_w), 8)
        pltpu.sync_copy(y_hbm.at[pl.ds(ybase, per_w)], y_v)
        pltpu.sync_copy(md_hbm, md_v)
        iota = lax.iota(jnp.int32, _L)
        zf = jnp.zeros((_L,), jnp.float32)
        onef = jnp.full((_L,), 1.0, jnp.float32)

        def zero_body(j, _):
            sbin_v[pl.ds(j * _L, _L)] = zf
            cbin_v[pl.ds(j * _L, _L)] = zf
            return 0
        lax.fori_loop(0, _L * c // _L, zero_body, 0)

        def r0_of(k):
            r0 = jnp.where(k < nfull, base + k * _B, base + rows_w - _B)
            return pl.multiple_of(r0, 8)

        def copy_of(k, slot):
            return pltpu.make_async_copy(
                close_hbm.at[pl.ds(r0_of(k), _B), :], blk_v.at[slot], sems[slot])

        def start(k):
            for slot in range(_NBUF):
                @pl.when(k % _NBUF == slot)
                def _():
                    copy_of(k, slot).start()

        for b in range(_NBUF - 1):
            @pl.when(b < nblk)
            def _():
                copy_of(b, b).start()

        def blk_body(k, _):
            @pl.when(k + _NBUF - 1 < nblk)
            def _():
                start(k + _NBUF - 1)
            for slot in range(_NBUF):
                @pl.when(k % _NBUF == slot)
                def _():
                    copy_of(k, slot).wait()
            r0 = r0_of(k)
            buf = blk_v.at[k % _NBUF]
            for q in range(_B // _L):
                row = r0 + q * _L + iota
                keep = row >= jnp.where(k < nfull, 0, thresh)
                yv = y_v[pl.ds(r0 - ybase + q * _L, _L)]
                rl = q * _L + iota
                g = plsc.load_gather(buf, [rl, yv])
                md = plsc.load_gather(md_v, [yv])
                ps = _per_sample_loss(md - g)
                bidx = iota * c + yv
                plsc.addupdate_scatter(sbin_v, [bidx], ps, mask=keep)
                plsc.addupdate_scatter(cbin_v, [bidx], onef, mask=keep)
            return 0
        lax.fori_loop(0, nblk, blk_body, 0)

        def fold_body(b, _):
            accs = zf
            accc = zf
            for l in range(_L):
                accs = accs + sbin_v[pl.ds(l * c + b * _L, _L)]
                accc = accc + cbin_v[pl.ds(l * c + b * _L, _L)]
            fold_v[pl.ds(b * _L, _L)] = accs
            fold_v[pl.ds(c + b * _L, _L)] = accc
            return 0
        lax.fori_loop(0, cb, fold_body, 0)
        pltpu.sync_copy(fold_v, part_hbm.at[pl.ds(wid * 2 * c, 2 * c)])

    @functools.partial(
        pl.kernel,
        out_type=jax.ShapeDtypeStruct((_L,), jnp.float32),
        mesh=mesh,
        compiler_params=params,
        scratch_types=[
            pltpu.VMEM((_NW * 2 * c,), jnp.float32),
            pltpu.VMEM((_L,), jnp.float32),
        ],
    )
    def k2(part_hbm, out_hbm, buf_v, out_v):
        wid = lax.axis_index("s") * _NC + lax.axis_index("c")

        @pl.when(wid == 0)
        def _():
            pltpu.sync_copy(part_hbm, buf_v)
            zf = jnp.zeros((_L,), jnp.float32)
            onef = jnp.full((_L,), 1.0, jnp.float32)

            def red_body(b, car):
                macc, jacc = car
                accs = zf
                accc = zf
                for w in range(_NW):
                    accs = accs + buf_v[pl.ds(w * 2 * c + b * _L, _L)]
                    accc = accc + buf_v[pl.ds(w * 2 * c + c + b * _L, _L)]
                ne = accc > 0.0
                mean = jnp.where(ne, accs / jnp.maximum(accc, 1.0), zf)
                return macc + mean, jacc + jnp.where(ne, onef, zf)

            macc, jacc = lax.fori_loop(0, cb, red_body, (zf, zf))
            out_v[...] = (zf + jnp.sum(macc)) / jnp.maximum(zf + jnp.sum(jacc), 1.0)
            pltpu.sync_copy(out_v, out_hbm)

    def run(close_er, y, max_dis, margin):
        md_plus = max_dis + margin.astype(jnp.float32)
        part = k1(close_er, y.astype(jnp.int32), md_plus)
        return k2(part)[0]

    return run


def kernel(close_er, y, max_dis, margin):
    n, c = close_er.shape
    return _build(n, c)(close_er, y, max_dis, jnp.asarray(margin))


# speedup vs baseline: 12.6383x; 1.0250x over previous
import functools

import jax
import jax.numpy as jnp
from jax import lax
from jax.experimental import pallas as pl
from jax.experimental.pallas import tpu as pltpu
from jax.experimental.pallas import tpu_sc as plsc

_NC = 2
_NS = 16
_NW = _NC * _NS
_L = 16
_B = 64
_NBUF = 6

_LO = 1.1920930376163597e-07
_HI = 16.118095651262775


def _per_sample_loss(u):
    e = jnp.exp(-jnp.abs(u))
    s = e / (e + 2.0)
    s2 = s * s
    p = 2.0 * s * (1.0 + s2 * (1.0 / 3.0 + s2 * (0.2 + s2 * (1.0 / 7.0 + s2 * (1.0 / 9.0)))))
    ps = jnp.maximum(u, 0.0) + p
    return jnp.clip(ps, _LO, _HI)


@functools.lru_cache(maxsize=None)
def _build(n, c):
    per_w = -(-n // (_NW * 8)) * 8
    assert c % 128 == 0 and n % 8 == 0
    assert n - (_NW - 1) * per_w >= _B
    cb = c // _L
    mesh = plsc.VectorSubcoreMesh(core_axis_name="c", subcore_axis_name="s")
    params = pltpu.CompilerParams(
        use_tc_tiling_on_sc=True, needs_layout_passes=False)

    @functools.partial(
        pl.kernel,
        out_type=jax.ShapeDtypeStruct((_NW * 2 * c,), jnp.float32),
        mesh=mesh,
        compiler_params=params,
        scratch_types=[
            pltpu.VMEM((per_w,), jnp.int32),
            pltpu.VMEM((_NBUF, _B, c), jnp.float32),
            pltpu.VMEM((c,), jnp.float32),
            pltpu.VMEM((_L * c,), jnp.float32),
            pltpu.VMEM((_L * c,), jnp.float32),
            pltpu.VMEM((2 * c,), jnp.float32),
            [pltpu.SemaphoreType.DMA] * _NBUF,
        ],
    )
    def k1(close_hbm, y_hbm, md_hbm, part_hbm,
           y_v, blk_v, md_v, sbin_v, cbin_v, fold_v, sems):
        wid = lax.axis_index("s") * _NC + lax.axis_index("c")
        base = wid * per_w
        rows_w = jnp.minimum(per_w, n - base)
        nfull = rows_w // _B
        thresh = base + nfull * _B
        nblk = nfull + 1
        ybase = pl.multiple_of(jnp.minimum(base, n - per_w), 8)
        pltpu.sync_copy(y_hbm.at[pl.ds(ybase, per_w)], y_v)
        pltpu.sync_copy(md_hbm, md_v)
        iota = lax.iota(jnp.int32, _L)
        zf = jnp.zeros((_L,), jnp.float32)
        onef = jnp.full((_L,), 1.0, jnp.float32)

        def zero_body(j, _):
            sbin_v[pl.ds(j * _L, _L)] = zf
            cbin_v[pl.ds(j * _L, _L)] = zf
            return 0
        lax.fori_loop(0, _L * c // _L, zero_body, 0)

        def r0_of(k):
            r0 = jnp.where(k < nfull, base + k * _B, base + rows_w - _B)
            return pl.multiple_of(r0, 8)

        def copy_of(k, slot):
            return pltpu.make_async_copy(
                close_hbm.at[pl.ds(r0_of(k), _B), :], blk_v.at[slot], sems[slot])

        def start(k):
            for slot in range(_NBUF):
                @pl.when(k % _NBUF == slot)
                def _():
                    copy_of(k, slot).start()

        for b in range(_NBUF - 1):
            @pl.when(b < nblk)
            def _():
                copy_of(b, b).start()

        def blk_body(k, _):
            @pl.when(k + _NBUF - 1 < nblk)
            def _():
                start(k + _NBUF - 1)
            for slot in range(_NBUF):
                @pl.when(k % _NBUF == slot)
                def _():
                    copy_of(k, slot).wait()
            r0 = r0_of(k)
            buf = blk_v.at[k % _NBUF]
            for q in range(_B // _L):
                row = r0 + q * _L + iota
                keep = row >= jnp.where(k < nfull, 0, thresh)
                yv = y_v[pl.ds(r0 - ybase + q * _L, _L)]
                rl = q * _L + iota
                g = plsc.load_gather(buf, [rl, yv])
                md = plsc.load_gather(md_v, [yv])
                ps = _per_sample_loss(md - g)
                bidx = iota * c + yv
                plsc.addupdate_scatter(sbin_v, [bidx], ps, mask=keep)
                plsc.addupdate_scatter(cbin_v, [bidx], onef, mask=keep)
            return 0
        lax.fori_loop(0, nblk, blk_body, 0)

        def fold_body(b, _):
            accs = zf
            accc = zf
            for l in range(_L):
                accs = accs + sbin_v[pl.ds(l * c + b * _L, _L)]
                accc = accc + cbin_v[pl.ds(l * c + b * _L, _L)]
            fold_v[pl.ds(b * _L, _L)] = accs
            fold_v[pl.ds(c + b * _L, _L)] = accc
            return 0
        lax.fori_loop(0, cb, fold_body, 0)
        pltpu.sync_copy(fold_v, part_hbm.at[pl.ds(wid * 2 * c, 2 * c)])

    @functools.partial(
        pl.kernel,
        out_type=jax.ShapeDtypeStruct((_L,), jnp.float32),
        mesh=mesh,
        compiler_params=params,
        scratch_types=[
            pltpu.VMEM((_NW * 2 * c,), jnp.float32),
            pltpu.VMEM((_L,), jnp.float32),
        ],
    )
    def k2(part_hbm, out_hbm, buf_v, out_v):
        wid = lax.axis_index("s") * _NC + lax.axis_index("c")

        @pl.when(wid == 0)
        def _():
            pltpu.sync_copy(part_hbm, buf_v)
            zf = jnp.zeros((_L,), jnp.float32)
            onef = jnp.full((_L,), 1.0, jnp.float32)

            def red_body(b, car):
                macc, jacc = car
                accs = zf
                accc = zf
                for w in range(_NW):
                    accs = accs + buf_v[pl.ds(w * 2 * c + b * _L, _L)]
                    accc = accc + buf_v[pl.ds(w * 2 * c + c + b * _L, _L)]
                ne = accc > 0.0
                mean = jnp.where(ne, accs / jnp.maximum(accc, 1.0), zf)
                return macc + mean, jacc + jnp.where(ne, onef, zf)

            macc, jacc = lax.fori_loop(0, cb, red_body, (zf, zf))
            out_v[...] = (zf + jnp.sum(macc)) / jnp.maximum(zf + jnp.sum(jacc), 1.0)
            pltpu.sync_copy(out_v, out_hbm)

    def run(close_er, y, max_dis, margin):
        md_plus = max_dis + margin.astype(jnp.float32)
        part = k1(close_er, y.astype(jnp.int32), md_plus)
        return k2(part)[0]

    return run


def kernel(close_er, y, max_dis, margin):
    n, c = close_er.shape
    return _build(n, c)(close_er, y, max_dis, jnp.asarray(margin))
